# Initial kernel scaffold; baseline (speedup 1.0000x reference)
import jax
import jax.numpy as jnp
from jax.experimental import pallas as pl


def kernel(x, table):
    raise NotImplementedError("write your pallas kernel here")


# speedup vs baseline: 10.4329x; 10.4329x over previous
import functools

import jax
import jax.numpy as jnp
from jax import lax
from jax.experimental import pallas as pl
from jax.experimental.pallas import tpu as pltpu
from jax.experimental.pallas import tpu_sc as plsc

NC = 2
NS = 16
NW = NC * NS


@functools.lru_cache(maxsize=None)
def _make_gather(n, vocab, d, chunk):
    per_w = n // NW
    n_chunks = per_w // chunk
    mesh = plsc.VectorSubcoreMesh(
        core_axis_name="c", subcore_axis_name="s",
        num_cores=NC, num_subcores=NS,
    )

    @functools.partial(
        pl.kernel,
        out_type=jax.ShapeDtypeStruct((n, d), jnp.float32),
        mesh=mesh,
        scratch_types=[
            pltpu.VMEM((chunk,), jnp.int32),
            pltpu.VMEM((chunk, d), jnp.float32),
            pltpu.SemaphoreType.DMA,
        ],
        compiler_params=pltpu.CompilerParams(use_tc_tiling_on_sc=False),
    )
    def gather(x_hbm, table_hbm, out_hbm, idx_v, rows_v, sem):
        wid = lax.axis_index("s") * NC + lax.axis_index("c")
        base = wid * per_w

        @pl.loop(0, n_chunks)
        def _(i):
            off = base + i * chunk
            pltpu.sync_copy(x_hbm.at[pl.ds(off, chunk)], idx_v)
            pltpu.async_copy(table_hbm.at[idx_v], rows_v, sem).wait()
            pltpu.sync_copy(rows_v, out_hbm.at[pl.ds(off, chunk)])

    return gather


@jax.jit
def kernel(x, table):
    b, t = x.shape
    vocab, d = table.shape
    xf = x.reshape(-1).astype(jnp.int32)
    out = _make_gather(b * t, vocab, d, 12800)(xf, table)
    return out.reshape(b, t, d)


# speedup vs baseline: 10.4441x; 1.0011x over previous
import functools

import jax
import jax.numpy as jnp
from jax import lax
from jax.experimental import pallas as pl
from jax.experimental.pallas import tpu as pltpu
from jax.experimental.pallas import tpu_sc as plsc

NC = 2
NS = 16
NW = NC * NS
NBUF = 2


@functools.lru_cache(maxsize=None)
def _make_gather(n, vocab, d, chunk):
    per_w = n // NW
    n_chunks = per_w // chunk
    assert n_chunks % NBUF == 0 and per_w % chunk == 0
    mesh = plsc.VectorSubcoreMesh(
        core_axis_name="c", subcore_axis_name="s",
        num_cores=NC, num_subcores=NS,
    )

    @functools.partial(
        pl.kernel,
        out_type=jax.ShapeDtypeStruct((n, d), jnp.float32),
        mesh=mesh,
        scratch_types=[
            pltpu.VMEM((NBUF, chunk), jnp.int32),
            pltpu.VMEM((NBUF, chunk, d), jnp.float32),
            pltpu.SemaphoreType.DMA((NBUF,)),
            pltpu.SemaphoreType.DMA((NBUF,)),
            pltpu.SemaphoreType.DMA((NBUF,)),
        ],
        compiler_params=pltpu.CompilerParams(use_tc_tiling_on_sc=False),
    )
    def gather(x_hbm, table_hbm, out_hbm, idx_v, rows_v, isem, gsem, osem):
        wid = lax.axis_index("s") * NC + lax.axis_index("c")
        base = wid * per_w

        def idx_copy(i, b):
            return pltpu.make_async_copy(
                x_hbm.at[pl.ds(base + i * chunk, chunk)], idx_v.at[b], isem.at[b])

        def gather_copy(b):
            return pltpu.make_async_copy(
                table_hbm.at[idx_v.at[b]], rows_v.at[b], gsem.at[b])

        def store_copy(i, b):
            return pltpu.make_async_copy(
                rows_v.at[b], out_hbm.at[pl.ds(base + i * chunk, chunk)], osem.at[b])

        for b in range(NBUF):
            idx_copy(b, b).start()

        @pl.loop(0, n_chunks, step=NBUF)
        def _(i0):
            for b in range(NBUF):
                i = i0 + b
                idx_copy(i, b).wait()

                @pl.when(i0 >= NBUF)
                def _():
                    store_copy(i, b).wait()

                gather_copy(b).start()
                gather_copy(b).wait()
                store_copy(i, b).start()

                @pl.when(i0 < n_chunks - NBUF)
                def _():
                    idx_copy(i + NBUF, b).start()

        for b in range(NBUF):
            store_copy(n_chunks - NBUF + b, b).wait()

    return gather


@jax.jit
def kernel(x, table):
    b, t = x.shape
    vocab, d = table.shape
    xf = x.reshape(-1).astype(jnp.int32)
    out = _make_gather(b * t, vocab, d, 6400)(xf, table)
    return out.reshape(b, t, d)
